# baseline (speedup 1.0000x reference)
import jax
import jax.numpy as jnp
from jax.experimental import pallas as pl
from jax.experimental.pallas import tpu as pltpu

E = 10
B = 16
GB = 8
CIN = 96
COUT = 96
H = 28
W = 28
HO = H - 2
WO = W - 2
NPIX = H * W
NROW = HO * W
NMM = NROW - 2


def _moe_kernel(x_ref, w1_ref, b1_ref, w2_ref, b2_ref, emb_ref, wt_ref,
                bc_ref, out_ref):
    embv = emb_ref[:, :]
    ssum = jnp.sum(embv * embv, axis=1, keepdims=True)
    enorm = jnp.sqrt(ssum) + 1e-8
    for b in range(GB):
        xb = x_ref[b]
        rc = jnp.sum(xb, axis=1, keepdims=True) * (1.0 / NPIX)
        h1 = jnp.maximum(
            jnp.dot(w1_ref[:, :], rc, preferred_element_type=jnp.float32)
            + b1_ref[:, :], 0.0)
        r2 = (jnp.dot(w2_ref[:, :], h1, preferred_element_type=jnp.float32)
              + b2_ref[:, :])
        d = jnp.dot(embv, r2, preferred_element_type=jnp.float32)
        sim = d / enorm
        e_idx = jnp.argmax(sim)

        xbh = xb.astype(jnp.bfloat16)
        xt = jnp.transpose(xbh)
        acc0 = jnp.broadcast_to(bc_ref[e_idx], (NMM, COUT))
        acc1 = jnp.zeros((NMM, COUT), jnp.float32)
        acc2 = jnp.zeros((NMM, COUT), jnp.float32)
        accs = [acc0, acc1, acc2]
        for di in range(3):
            for dj in range(3):
                base = di * W + dj
                fs = xt[base:base + NMM, :]
                wk = wt_ref[e_idx, di, dj]
                k = (di * 3 + dj) % 3
                accs[k] = accs[k] + jnp.dot(
                    fs, wk, preferred_element_type=jnp.float32)
        acc = (accs[0] + accs[1]) + accs[2]
        acc_t = jnp.transpose(acc)
        out_ref[b, :, :NMM] = acc_t


def kernel(x, W1, b1, W2, b2, emb, Wc, bc, task):
    x3 = x.reshape(B, CIN, NPIX)
    wt = jnp.transpose(Wc, (0, 3, 4, 2, 1)).astype(jnp.bfloat16)
    b1r = b1.reshape(128, 1)
    b2r = b2.reshape(64, 1)
    bcr = bc.reshape(E, 1, COUT)

    out = pl.pallas_call(
        _moe_kernel,
        grid=(B // GB,),
        compiler_params=pltpu.CompilerParams(
            dimension_semantics=("parallel",)),
        in_specs=[
            pl.BlockSpec((GB, CIN, NPIX), lambda i: (i, 0, 0)),
            pl.BlockSpec((128, CIN), lambda i: (0, 0)),
            pl.BlockSpec((128, 1), lambda i: (0, 0)),
            pl.BlockSpec((64, 128), lambda i: (0, 0)),
            pl.BlockSpec((64, 1), lambda i: (0, 0)),
            pl.BlockSpec((E, 64), lambda i: (0, 0)),
            pl.BlockSpec((E, 3, 3, CIN, COUT), lambda i: (0, 0, 0, 0, 0)),
            pl.BlockSpec((E, 1, COUT), lambda i: (0, 0, 0)),
        ],
        out_specs=pl.BlockSpec((GB, COUT, NROW), lambda i: (i, 0, 0)),
        out_shape=jax.ShapeDtypeStruct((B, COUT, NROW), jnp.float32),
    )(x3, W1, b1r, W2, b2r, emb, wt, bcr)

    return out.reshape(B, COUT, HO, W)[:, :, :, :WO]


# speedup vs baseline: 1.0165x; 1.0165x over previous
import jax
import jax.numpy as jnp
from jax.experimental import pallas as pl

E = 10
B = 16
CIN = 96
COUT = 96
H = 28
W = 28
HO = H - 2
WO = W - 2
NPIX = H * W
NROW = HO * W
NMM = NROW - 2


def _moe_kernel(x_ref, w1_ref, b1_ref, w2_ref, b2_ref, emb_ref, wt_ref,
                bc_ref, out_ref):
    embv = emb_ref[:, :]
    ssum = jnp.sum(embv * embv, axis=1, keepdims=True)
    en = (embv / (jnp.sqrt(ssum) + 1e-8)).astype(jnp.bfloat16)
    for b in range(B):
        xb = x_ref[b]
        rc = jnp.sum(xb, axis=1, keepdims=True) * (1.0 / NPIX)
        h1 = jnp.maximum(
            jnp.dot(w1_ref[:, :].astype(jnp.bfloat16),
                    rc.astype(jnp.bfloat16),
                    preferred_element_type=jnp.float32)
            + b1_ref[:, :], 0.0)
        r2 = (jnp.dot(w2_ref[:, :].astype(jnp.bfloat16),
                      h1.astype(jnp.bfloat16),
                      preferred_element_type=jnp.float32)
              + b2_ref[:, :])
        rnorm = jnp.sqrt(jnp.sum(r2 * r2, axis=0, keepdims=True)) + 1e-8
        rn = (r2 / rnorm).astype(jnp.bfloat16)
        sim = jnp.dot(en, rn, preferred_element_type=jnp.float32)
        e_idx = jnp.argmax(sim)

        xbh = xb.astype(jnp.bfloat16)
        xt = jnp.transpose(xbh)
        acc0 = jnp.broadcast_to(bc_ref[e_idx], (NMM, COUT))
        acc1 = jnp.zeros((NMM, COUT), jnp.float32)
        acc2 = jnp.zeros((NMM, COUT), jnp.float32)
        accs = [acc0, acc1, acc2]
        for di in range(3):
            for dj in range(3):
                base = di * W + dj
                fs = xt[base:base + NMM, :]
                wk = wt_ref[e_idx, di, dj]
                k = (di * 3 + dj) % 3
                accs[k] = accs[k] + jnp.dot(
                    fs, wk, preferred_element_type=jnp.float32)
        acc = (accs[0] + accs[1]) + accs[2]
        acc_t = jnp.transpose(acc)
        out_ref[b, :, :NMM] = acc_t


def kernel(x, W1, b1, W2, b2, emb, Wc, bc, task):
    x3 = x.reshape(B, CIN, NPIX)
    wt = jnp.transpose(Wc, (0, 3, 4, 2, 1)).astype(jnp.bfloat16)
    b1r = b1.reshape(128, 1)
    b2r = b2.reshape(64, 1)
    bcr = bc.reshape(E, 1, COUT)

    out = pl.pallas_call(
        _moe_kernel,
        grid=(1,),
        in_specs=[
            pl.BlockSpec((B, CIN, NPIX), lambda i: (0, 0, 0)),
            pl.BlockSpec((128, CIN), lambda i: (0, 0)),
            pl.BlockSpec((128, 1), lambda i: (0, 0)),
            pl.BlockSpec((64, 128), lambda i: (0, 0)),
            pl.BlockSpec((64, 1), lambda i: (0, 0)),
            pl.BlockSpec((E, 64), lambda i: (0, 0)),
            pl.BlockSpec((E, 3, 3, CIN, COUT), lambda i: (0, 0, 0, 0, 0)),
            pl.BlockSpec((E, 1, COUT), lambda i: (0, 0, 0)),
        ],
        out_specs=pl.BlockSpec((B, COUT, NROW), lambda i: (0, 0, 0)),
        out_shape=jax.ShapeDtypeStruct((B, COUT, NROW), jnp.float32),
    )(x3, W1, b1r, W2, b2r, emb, wt, bcr)

    return out.reshape(B, COUT, HO, W)[:, :, :, :WO]
